# baseline (speedup 1.0000x reference)
import numpy as np
import jax
import jax.numpy as jnp
from jax.experimental import pallas as pl
from jax.experimental.pallas import tpu as pltpu

_B, _C, _H, _W = 4, 96, 384, 384
_PH, _PW = _H // 2, _W // 2
_PB = 48
_GROWS = _PH // _PB
_NREG = _PH * _PW
_LW = 5 * _PW

_PE_NP = np.zeros((_W, _PW), np.float32)
_PE_NP[np.arange(0, _W, 2), np.arange(_PW)] = 1.0
_PO_NP = np.zeros((_W, _PW), np.float32)
_PO_NP[np.arange(1, _W, 2), np.arange(_PW)] = 1.0


def _rotl(x, r):
    return (x << jnp.uint32(r)) | (x >> jnp.uint32(32 - r))


def _threefry_bits(k0, k1, ctr):
    ks2 = k0 ^ k1 ^ jnp.uint32(0x1BD11BDA)
    ks = (k0, k1, ks2)
    x0 = jnp.zeros(ctr.shape, jnp.uint32) + k0
    x1 = ctr + k1
    rots = ((13, 15, 26, 6), (17, 29, 16, 24))
    for i in range(5):
        for r in rots[i % 2]:
            x0 = x0 + x1
            x1 = _rotl(x1, r)
            x1 = x1 ^ x0
        x0 = x0 + ks[(i + 1) % 3]
        x1 = x1 + ks[(i + 2) % 3] + jnp.uint32(i + 1)
    return x0 ^ x1


def _uniform(bits):
    f = jax.lax.bitcast_convert_type((bits >> jnp.uint32(9)) | jnp.uint32(0x3F800000),
                                     jnp.float32)
    return jnp.maximum(jnp.float32(0.0), f - jnp.float32(1.0))


def _body(key_ref, x_ref, pe_ref, po_ref, sparse_ref, pooled_ref, winner_ref):
    k0 = key_ref[0].astype(jnp.uint32)
    k1 = key_ref[1].astype(jnp.uint32)

    b = pl.program_id(0)
    c = pl.program_id(1)
    g = pl.program_id(2)
    row = jax.lax.broadcasted_iota(jnp.int32, (_PB, _LW), 0)
    lane = jax.lax.broadcasted_iota(jnp.int32, (_PB, _LW), 1)
    pw = lane % _PW
    k_slot = lane // _PW
    ph = g * _PB + row
    ctr = (((b * _C + c) * _NREG) + ph * _PW + pw) * 5 + k_slot
    u = _uniform(_threefry_bits(k0, k1, ctr.astype(jnp.uint32)))
    eps = jnp.float32(1e-8)
    gum = -jnp.log(-jnp.log(u + eps) + eps)

    pe = pe_ref[...]
    po = po_ref[...]

    def dot(a, b_):
        return jax.lax.dot_general(a, b_, (((1,), (0,)), ((), ())),
                                   preferred_element_type=jnp.float32)

    def dot_t(a, b_):
        return jax.lax.dot_general(a, b_, (((1,), (1,)), ((), ())),
                                   preferred_element_type=jnp.float32)

    def deinterleave(r, sel):
        hi = r.astype(jnp.bfloat16)
        rem = r - hi.astype(jnp.float32)
        lo = rem.astype(jnp.bfloat16)
        lo2 = (rem - lo.astype(jnp.float32)).astype(jnp.bfloat16)
        return (dot(hi, sel) + dot(lo, sel)) + dot(lo2, sel)

    xb = x_ref[0, 0]
    r0 = xb[:, :_W]
    r1 = xb[:, _W:]
    v0 = deinterleave(r0, pe)
    v1 = deinterleave(r0, po)
    v2 = deinterleave(r1, pe)
    v3 = deinterleave(r1, po)

    m = jnp.maximum(jnp.maximum(jnp.maximum(v0, v1), jnp.maximum(v2, v3)),
                    jnp.float32(0.0))
    e0 = jnp.exp(v0 - m)
    e1 = jnp.exp(v1 - m)
    e2 = jnp.exp(v2 - m)
    e3 = jnp.exp(v3 - m)
    e4 = jnp.exp(-m)
    inv = jnp.float32(1.0) / (e0 + e1 + e2 + e3 + e4 + eps)
    probs = (e0 * inv, e1 * inv, e2 * inv, e3 * inv, e4 * inv)

    best = None
    w = None
    for k in range(5):
        s = jnp.log(probs[k] + eps) + gum[:, k * _PW:(k + 1) * _PW]
        if k == 0:
            best = s
            w = jnp.zeros((_PB, _PW), jnp.int32)
        else:
            take = s > best
            w = jnp.where(take, jnp.int32(k), w)
            best = jnp.maximum(best, s)

    sv = [jnp.where(w == k, probs[k], jnp.float32(0.0)).astype(jnp.bfloat16)
          for k in range(4)]
    sparse_ref[0, 0, :, :_W] = dot_t(sv[0], pe) + dot_t(sv[1], po)
    sparse_ref[0, 0, :, _W:] = dot_t(sv[2], pe) + dot_t(sv[3], po)
    pooled_ref[0, 0] = (e0 + e1 + e2 + e3) * inv
    winner_ref[0, 0] = w


@jax.jit
def kernel(hidden_activations):
    x = hidden_activations.reshape(_B, _C, _PH, 2 * _W)
    key_data = jax.random.key_data(
        jax.random.fold_in(jax.random.key(0), 7)).astype(jnp.int32)

    grid_spec = pltpu.PrefetchScalarGridSpec(
        num_scalar_prefetch=1,
        grid=(_B, _C, _GROWS),
        in_specs=[
            pl.BlockSpec((1, 1, _PB, 2 * _W), lambda b, c, g, k: (b, c, g, 0)),
            pl.BlockSpec((_W, _PW), lambda b, c, g, k: (0, 0)),
            pl.BlockSpec((_W, _PW), lambda b, c, g, k: (0, 0)),
        ],
        out_specs=[
            pl.BlockSpec((1, 1, _PB, 2 * _W), lambda b, c, g, k: (b, c, g, 0)),
            pl.BlockSpec((1, 1, _PB, _PW), lambda b, c, g, k: (b, c, g, 0)),
            pl.BlockSpec((1, 1, _PB, _PW), lambda b, c, g, k: (b, c, g, 0)),
        ],
    )
    sparse768, pooled, winner = pl.pallas_call(
        _body,
        grid_spec=grid_spec,
        out_shape=[
            jax.ShapeDtypeStruct((_B, _C, _PH, 2 * _W), jnp.float32),
            jax.ShapeDtypeStruct((_B, _C, _PH, _PW), jnp.float32),
            jax.ShapeDtypeStruct((_B, _C, _PH, _PW), jnp.int32),
        ],
    )(key_data, x, jnp.asarray(_PE_NP).astype(jnp.bfloat16),
      jnp.asarray(_PO_NP).astype(jnp.bfloat16))

    sparse = sparse768.reshape(_B, _C, _H, _W)
    return (sparse, pooled, winner)


# speedup vs baseline: 1.1969x; 1.1969x over previous
import numpy as np
import jax
import jax.numpy as jnp
from jax.experimental import pallas as pl
from jax.experimental.pallas import tpu as pltpu

_B, _C, _H, _W = 4, 96, 384, 384
_PH, _PW = _H // 2, _W // 2
_PB = 96
_GROWS = _PH // _PB
_NREG = _PH * _PW
_LW = 5 * _PW

_PE_NP = np.zeros((_W, _PW), np.float32)
_PE_NP[np.arange(0, _W, 2), np.arange(_PW)] = 1.0
_PO_NP = np.zeros((_W, _PW), np.float32)
_PO_NP[np.arange(1, _W, 2), np.arange(_PW)] = 1.0


def _rotl(x, r):
    return (x << jnp.uint32(r)) | (x >> jnp.uint32(32 - r))


def _threefry_bits(k0, k1, ctr):
    ks2 = k0 ^ k1 ^ jnp.uint32(0x1BD11BDA)
    ks = (k0, k1, ks2)
    x0 = jnp.zeros(ctr.shape, jnp.uint32) + k0
    x1 = ctr + k1
    rots = ((13, 15, 26, 6), (17, 29, 16, 24))
    for i in range(5):
        for r in rots[i % 2]:
            x0 = x0 + x1
            x1 = _rotl(x1, r)
            x1 = x1 ^ x0
        x0 = x0 + ks[(i + 1) % 3]
        x1 = x1 + ks[(i + 2) % 3] + jnp.uint32(i + 1)
    return x0 ^ x1


def _uniform(bits):
    f = jax.lax.bitcast_convert_type((bits >> jnp.uint32(9)) | jnp.uint32(0x3F800000),
                                     jnp.float32)
    return jnp.maximum(jnp.float32(0.0), f - jnp.float32(1.0))


def _body(key_ref, x_ref, pe_ref, po_ref, sparse_ref, pooled_ref, winner_ref):
    k0 = key_ref[0].astype(jnp.uint32)
    k1 = key_ref[1].astype(jnp.uint32)

    b = pl.program_id(0)
    c = pl.program_id(1)
    g = pl.program_id(2)
    row = jax.lax.broadcasted_iota(jnp.int32, (_PB, _LW), 0)
    lane = jax.lax.broadcasted_iota(jnp.int32, (_PB, _LW), 1)
    pw = lane % _PW
    k_slot = lane // _PW
    ph = g * _PB + row
    ctr = (((b * _C + c) * _NREG) + ph * _PW + pw) * 5 + k_slot
    u = _uniform(_threefry_bits(k0, k1, ctr.astype(jnp.uint32)))
    eps = jnp.float32(1e-8)
    gum = -jnp.log(-jnp.log(u + eps) + eps)

    pe = pe_ref[...]
    po = po_ref[...]

    def dot(a, b_):
        return jax.lax.dot_general(a, b_, (((1,), (0,)), ((), ())),
                                   preferred_element_type=jnp.float32)

    def dot_t(a, b_):
        return jax.lax.dot_general(a, b_, (((1,), (1,)), ((), ())),
                                   preferred_element_type=jnp.float32)

    def deinterleave(r, sel):
        hi = r.astype(jnp.bfloat16)
        rem = r - hi.astype(jnp.float32)
        lo = rem.astype(jnp.bfloat16)
        lo2 = (rem - lo.astype(jnp.float32)).astype(jnp.bfloat16)
        return (dot(hi, sel) + dot(lo, sel)) + dot(lo2, sel)

    xb = x_ref[0, 0]
    r0 = xb[:, :_W]
    r1 = xb[:, _W:]
    v0 = deinterleave(r0, pe)
    v1 = deinterleave(r0, po)
    v2 = deinterleave(r1, pe)
    v3 = deinterleave(r1, po)

    m = jnp.maximum(jnp.maximum(jnp.maximum(v0, v1), jnp.maximum(v2, v3)),
                    jnp.float32(0.0))
    e0 = jnp.exp(v0 - m)
    e1 = jnp.exp(v1 - m)
    e2 = jnp.exp(v2 - m)
    e3 = jnp.exp(v3 - m)
    e4 = jnp.exp(-m)
    inv = jnp.float32(1.0) / (e0 + e1 + e2 + e3 + e4 + eps)
    probs = (e0 * inv, e1 * inv, e2 * inv, e3 * inv, e4 * inv)

    best = None
    w = None
    for k in range(5):
        s = jnp.log(probs[k] + eps) + gum[:, k * _PW:(k + 1) * _PW]
        if k == 0:
            best = s
            w = jnp.zeros((_PB, _PW), jnp.int32)
        else:
            take = s > best
            w = jnp.where(take, jnp.int32(k), w)
            best = jnp.maximum(best, s)

    sv = [jnp.where(w == k, probs[k], jnp.float32(0.0)).astype(jnp.bfloat16)
          for k in range(4)]
    sparse_ref[0, 0, :, :_W] = dot_t(sv[0], pe) + dot_t(sv[1], po)
    sparse_ref[0, 0, :, _W:] = dot_t(sv[2], pe) + dot_t(sv[3], po)
    pooled_ref[0, 0] = (e0 + e1 + e2 + e3) * inv
    winner_ref[0, 0] = w


@jax.jit
def kernel(hidden_activations):
    x = hidden_activations.reshape(_B, _C, _PH, 2 * _W)
    key_data = jax.random.key_data(
        jax.random.fold_in(jax.random.key(0), 7)).astype(jnp.int32)

    grid_spec = pltpu.PrefetchScalarGridSpec(
        num_scalar_prefetch=1,
        grid=(_B, _C, _GROWS),
        in_specs=[
            pl.BlockSpec((1, 1, _PB, 2 * _W), lambda b, c, g, k: (b, c, g, 0)),
            pl.BlockSpec((_W, _PW), lambda b, c, g, k: (0, 0)),
            pl.BlockSpec((_W, _PW), lambda b, c, g, k: (0, 0)),
        ],
        out_specs=[
            pl.BlockSpec((1, 1, _PB, 2 * _W), lambda b, c, g, k: (b, c, g, 0)),
            pl.BlockSpec((1, 1, _PB, _PW), lambda b, c, g, k: (b, c, g, 0)),
            pl.BlockSpec((1, 1, _PB, _PW), lambda b, c, g, k: (b, c, g, 0)),
        ],
    )
    sparse768, pooled, winner = pl.pallas_call(
        _body,
        grid_spec=grid_spec,
        out_shape=[
            jax.ShapeDtypeStruct((_B, _C, _PH, 2 * _W), jnp.float32),
            jax.ShapeDtypeStruct((_B, _C, _PH, _PW), jnp.float32),
            jax.ShapeDtypeStruct((_B, _C, _PH, _PW), jnp.int32),
        ],
    )(key_data, x, jnp.asarray(_PE_NP).astype(jnp.bfloat16),
      jnp.asarray(_PO_NP).astype(jnp.bfloat16))

    sparse = sparse768.reshape(_B, _C, _H, _W)
    return (sparse, pooled, winner)


# speedup vs baseline: 1.3007x; 1.0867x over previous
import numpy as np
import jax
import jax.numpy as jnp
from jax.experimental import pallas as pl
from jax.experimental.pallas import tpu as pltpu

_B, _C, _H, _W = 4, 96, 384, 384
_PH, _PW = _H // 2, _W // 2
_PB = 192
_GROWS = _PH // _PB
_NREG = _PH * _PW
_LW = 5 * _PW

_PE_NP = np.zeros((_W, _PW), np.float32)
_PE_NP[np.arange(0, _W, 2), np.arange(_PW)] = 1.0
_PO_NP = np.zeros((_W, _PW), np.float32)
_PO_NP[np.arange(1, _W, 2), np.arange(_PW)] = 1.0


def _rotl(x, r):
    return (x << jnp.uint32(r)) | (x >> jnp.uint32(32 - r))


def _threefry_bits(k0, k1, ctr):
    ks2 = k0 ^ k1 ^ jnp.uint32(0x1BD11BDA)
    ks = (k0, k1, ks2)
    x0 = jnp.zeros(ctr.shape, jnp.uint32) + k0
    x1 = ctr + k1
    rots = ((13, 15, 26, 6), (17, 29, 16, 24))
    for i in range(5):
        for r in rots[i % 2]:
            x0 = x0 + x1
            x1 = _rotl(x1, r)
            x1 = x1 ^ x0
        x0 = x0 + ks[(i + 1) % 3]
        x1 = x1 + ks[(i + 2) % 3] + jnp.uint32(i + 1)
    return x0 ^ x1


def _uniform(bits):
    f = jax.lax.bitcast_convert_type((bits >> jnp.uint32(9)) | jnp.uint32(0x3F800000),
                                     jnp.float32)
    return jnp.maximum(jnp.float32(0.0), f - jnp.float32(1.0))


def _body(key_ref, x_ref, pe_ref, po_ref, sparse_ref, pooled_ref, winner_ref):
    k0 = key_ref[0].astype(jnp.uint32)
    k1 = key_ref[1].astype(jnp.uint32)

    b = pl.program_id(0)
    c = pl.program_id(1)
    g = pl.program_id(2)
    row = jax.lax.broadcasted_iota(jnp.int32, (_PB, _LW), 0)
    lane = jax.lax.broadcasted_iota(jnp.int32, (_PB, _LW), 1)
    pw = lane % _PW
    k_slot = lane // _PW
    ph = g * _PB + row
    ctr = (((b * _C + c) * _NREG) + ph * _PW + pw) * 5 + k_slot
    u = _uniform(_threefry_bits(k0, k1, ctr.astype(jnp.uint32)))
    eps = jnp.float32(1e-8)
    gum = -jnp.log(-jnp.log(u + eps) + eps)

    pe = pe_ref[...]
    po = po_ref[...]

    def dot(a, b_):
        return jax.lax.dot_general(a, b_, (((1,), (0,)), ((), ())),
                                   preferred_element_type=jnp.float32)

    def dot_t(a, b_):
        return jax.lax.dot_general(a, b_, (((1,), (1,)), ((), ())),
                                   preferred_element_type=jnp.float32)

    def deinterleave(r, sel):
        hi = r.astype(jnp.bfloat16)
        rem = r - hi.astype(jnp.float32)
        lo = rem.astype(jnp.bfloat16)
        lo2 = (rem - lo.astype(jnp.float32)).astype(jnp.bfloat16)
        return (dot(hi, sel) + dot(lo, sel)) + dot(lo2, sel)

    xb = x_ref[0, 0]
    r0 = xb[:, :_W]
    r1 = xb[:, _W:]
    v0 = deinterleave(r0, pe)
    v1 = deinterleave(r0, po)
    v2 = deinterleave(r1, pe)
    v3 = deinterleave(r1, po)

    m = jnp.maximum(jnp.maximum(jnp.maximum(v0, v1), jnp.maximum(v2, v3)),
                    jnp.float32(0.0))
    e0 = jnp.exp(v0 - m)
    e1 = jnp.exp(v1 - m)
    e2 = jnp.exp(v2 - m)
    e3 = jnp.exp(v3 - m)
    e4 = jnp.exp(-m)
    inv = jnp.float32(1.0) / (e0 + e1 + e2 + e3 + e4 + eps)
    probs = (e0 * inv, e1 * inv, e2 * inv, e3 * inv, e4 * inv)

    best = None
    w = None
    for k in range(5):
        s = jnp.log(probs[k] + eps) + gum[:, k * _PW:(k + 1) * _PW]
        if k == 0:
            best = s
            w = jnp.zeros((_PB, _PW), jnp.int32)
        else:
            take = s > best
            w = jnp.where(take, jnp.int32(k), w)
            best = jnp.maximum(best, s)

    sv = [jnp.where(w == k, probs[k], jnp.float32(0.0)).astype(jnp.bfloat16)
          for k in range(4)]
    sparse_ref[0, 0, :, :_W] = dot_t(sv[0], pe) + dot_t(sv[1], po)
    sparse_ref[0, 0, :, _W:] = dot_t(sv[2], pe) + dot_t(sv[3], po)
    pooled_ref[0, 0] = (e0 + e1 + e2 + e3) * inv
    winner_ref[0, 0] = w


@jax.jit
def kernel(hidden_activations):
    x = hidden_activations.reshape(_B, _C, _PH, 2 * _W)
    key_data = jax.random.key_data(
        jax.random.fold_in(jax.random.key(0), 7)).astype(jnp.int32)

    grid_spec = pltpu.PrefetchScalarGridSpec(
        num_scalar_prefetch=1,
        grid=(_B, _C, _GROWS),
        in_specs=[
            pl.BlockSpec((1, 1, _PB, 2 * _W), lambda b, c, g, k: (b, c, g, 0)),
            pl.BlockSpec((_W, _PW), lambda b, c, g, k: (0, 0)),
            pl.BlockSpec((_W, _PW), lambda b, c, g, k: (0, 0)),
        ],
        out_specs=[
            pl.BlockSpec((1, 1, _PB, 2 * _W), lambda b, c, g, k: (b, c, g, 0)),
            pl.BlockSpec((1, 1, _PB, _PW), lambda b, c, g, k: (b, c, g, 0)),
            pl.BlockSpec((1, 1, _PB, _PW), lambda b, c, g, k: (b, c, g, 0)),
        ],
    )
    sparse768, pooled, winner = pl.pallas_call(
        _body,
        grid_spec=grid_spec,
        out_shape=[
            jax.ShapeDtypeStruct((_B, _C, _PH, 2 * _W), jnp.float32),
            jax.ShapeDtypeStruct((_B, _C, _PH, _PW), jnp.float32),
            jax.ShapeDtypeStruct((_B, _C, _PH, _PW), jnp.int32),
        ],
    )(key_data, x, jnp.asarray(_PE_NP).astype(jnp.bfloat16),
      jnp.asarray(_PO_NP).astype(jnp.bfloat16))

    sparse = sparse768.reshape(_B, _C, _H, _W)
    return (sparse, pooled, winner)


# speedup vs baseline: 1.3176x; 1.0130x over previous
import numpy as np
import jax
import jax.numpy as jnp
from jax.experimental import pallas as pl
from jax.experimental.pallas import tpu as pltpu

_B, _C, _H, _W = 4, 96, 384, 384
_PH, _PW = _H // 2, _W // 2
_PB = 192
_GROWS = _PH // _PB
_NREG = _PH * _PW
_LW = 5 * _PW

_PE_NP = np.zeros((_W, _PW), np.float32)
_PE_NP[np.arange(0, _W, 2), np.arange(_PW)] = 1.0
_PO_NP = np.zeros((_W, _PW), np.float32)
_PO_NP[np.arange(1, _W, 2), np.arange(_PW)] = 1.0
_PAT_NP = ((np.arange(_LW) % _PW) * 5 + np.arange(_LW) // _PW).astype(np.int32)
_PAT_NP = _PAT_NP.reshape(1, _LW)


def _rotl(x, r):
    return (x << jnp.uint32(r)) | (x >> jnp.uint32(32 - r))


def _threefry_bits(k0, k1, ctr):
    ks2 = k0 ^ k1 ^ jnp.uint32(0x1BD11BDA)
    ks = (k0, k1, ks2)
    x0 = jnp.zeros(ctr.shape, jnp.uint32) + k0
    x1 = ctr + k1
    rots = ((13, 15, 26, 6), (17, 29, 16, 24))
    for i in range(5):
        for r in rots[i % 2]:
            x0 = x0 + x1
            x1 = _rotl(x1, r)
            x1 = x1 ^ x0
        x0 = x0 + ks[(i + 1) % 3]
        x1 = x1 + ks[(i + 2) % 3] + jnp.uint32(i + 1)
    return x0 ^ x1


def _uniform(bits):
    f = jax.lax.bitcast_convert_type((bits >> jnp.uint32(9)) | jnp.uint32(0x3F800000),
                                     jnp.float32)
    return jnp.maximum(jnp.float32(0.0), f - jnp.float32(1.0))


def _body(key_ref, x_ref, pe_ref, po_ref, pat_ref, sparse_ref, pooled_ref,
          winner_ref):
    k0 = key_ref[0].astype(jnp.uint32)
    k1 = key_ref[1].astype(jnp.uint32)

    b = pl.program_id(0)
    c = pl.program_id(1)
    g = pl.program_id(2)
    row = jax.lax.broadcasted_iota(jnp.int32, (_PB, _LW), 0)
    base = ((b * _C + c) * _NREG + g * _PB * _PW) * 5
    ctr = base + row * (5 * _PW) + pat_ref[...]
    u = _uniform(_threefry_bits(k0, k1, ctr.astype(jnp.uint32)))
    eps = jnp.float32(1e-8)
    gum = -jnp.log(-jnp.log(u + eps) + eps)

    pe = pe_ref[...]
    po = po_ref[...]

    def dot(a, b_):
        return jax.lax.dot_general(a, b_, (((1,), (0,)), ((), ())),
                                   preferred_element_type=jnp.float32)

    def dot_t(a, b_):
        return jax.lax.dot_general(a, b_, (((1,), (1,)), ((), ())),
                                   preferred_element_type=jnp.float32)

    def deinterleave(r, sel):
        hi = r.astype(jnp.bfloat16)
        rem = r - hi.astype(jnp.float32)
        lo = rem.astype(jnp.bfloat16)
        lo2 = (rem - lo.astype(jnp.float32)).astype(jnp.bfloat16)
        return dot(hi, sel) + (dot(lo, sel) + dot(lo2, sel))

    xb = x_ref[0, 0]
    r0 = xb[:, :_W]
    r1 = xb[:, _W:]
    v0 = deinterleave(r0, pe)
    v1 = deinterleave(r0, po)
    v2 = deinterleave(r1, pe)
    v3 = deinterleave(r1, po)

    m = jnp.maximum(jnp.maximum(jnp.maximum(v0, v1), jnp.maximum(v2, v3)),
                    jnp.float32(0.0))
    e0 = jnp.exp(v0 - m)
    e1 = jnp.exp(v1 - m)
    e2 = jnp.exp(v2 - m)
    e3 = jnp.exp(v3 - m)
    e4 = jnp.exp(-m)
    inv = jnp.float32(1.0) / (e0 + e1 + e2 + e3 + e4 + eps)
    probs = (e0 * inv, e1 * inv, e2 * inv, e3 * inv, e4 * inv)

    best = None
    w = None
    for k in range(5):
        s = jnp.log(probs[k] + eps) + gum[:, k * _PW:(k + 1) * _PW]
        if k == 0:
            best = s
            w = jnp.zeros((_PB, _PW), jnp.int32)
        else:
            take = s > best
            w = jnp.where(take, jnp.int32(k), w)
            best = jnp.maximum(best, s)

    sv = [jnp.where(w == k, probs[k], jnp.float32(0.0)).astype(jnp.bfloat16)
          for k in range(4)]
    sparse_ref[0, 0, :, :_W] = dot_t(sv[0], pe) + dot_t(sv[1], po)
    sparse_ref[0, 0, :, _W:] = dot_t(sv[2], pe) + dot_t(sv[3], po)
    pooled_ref[0, 0] = (e0 + e1 + e2 + e3) * inv
    winner_ref[0, 0] = w


@jax.jit
def kernel(hidden_activations):
    x = hidden_activations.reshape(_B, _C, _PH, 2 * _W)
    key_data = jax.random.key_data(
        jax.random.fold_in(jax.random.key(0), 7)).astype(jnp.int32)

    grid_spec = pltpu.PrefetchScalarGridSpec(
        num_scalar_prefetch=1,
        grid=(_B, _C, _GROWS),
        in_specs=[
            pl.BlockSpec((1, 1, _PB, 2 * _W), lambda b, c, g, k: (b, c, g, 0)),
            pl.BlockSpec((_W, _PW), lambda b, c, g, k: (0, 0)),
            pl.BlockSpec((_W, _PW), lambda b, c, g, k: (0, 0)),
            pl.BlockSpec((1, _LW), lambda b, c, g, k: (0, 0)),
        ],
        out_specs=[
            pl.BlockSpec((1, 1, _PB, 2 * _W), lambda b, c, g, k: (b, c, g, 0)),
            pl.BlockSpec((1, 1, _PB, _PW), lambda b, c, g, k: (b, c, g, 0)),
            pl.BlockSpec((1, 1, _PB, _PW), lambda b, c, g, k: (b, c, g, 0)),
        ],
    )
    sparse768, pooled, winner = pl.pallas_call(
        _body,
        grid_spec=grid_spec,
        out_shape=[
            jax.ShapeDtypeStruct((_B, _C, _PH, 2 * _W), jnp.float32),
            jax.ShapeDtypeStruct((_B, _C, _PH, _PW), jnp.float32),
            jax.ShapeDtypeStruct((_B, _C, _PH, _PW), jnp.int32),
        ],
    )(key_data, x, jnp.asarray(_PE_NP).astype(jnp.bfloat16),
      jnp.asarray(_PO_NP).astype(jnp.bfloat16), jnp.asarray(_PAT_NP))

    sparse = sparse768.reshape(_B, _C, _H, _W)
    return (sparse, pooled, winner)


# speedup vs baseline: 1.3591x; 1.0315x over previous
import numpy as np
import jax
import jax.numpy as jnp
from jax.experimental import pallas as pl
from jax.experimental.pallas import tpu as pltpu

_B, _C, _H, _W = 4, 96, 384, 384
_PH, _PW = _H // 2, _W // 2
_PB = 192
_GROWS = _PH // _PB
_NREG = _PH * _PW
_LW = 5 * _PW

_PE_NP = np.zeros((_W, _PW), np.float32)
_PE_NP[np.arange(0, _W, 2), np.arange(_PW)] = 1.0
_PO_NP = np.zeros((_W, _PW), np.float32)
_PO_NP[np.arange(1, _W, 2), np.arange(_PW)] = 1.0
_PAT_NP = ((np.arange(_LW) % _PW) * 5 + np.arange(_LW) // _PW).astype(np.int32)
_PAT_NP = _PAT_NP.reshape(1, _LW)


def _rotl(x, r):
    return (x << jnp.uint32(r)) | (x >> jnp.uint32(32 - r))


def _threefry_bits(k0, k1, ctr):
    ks2 = k0 ^ k1 ^ jnp.uint32(0x1BD11BDA)
    ks = (k0, k1, ks2)
    x0 = jnp.zeros(ctr.shape, jnp.uint32) + k0
    x1 = ctr + k1
    rots = ((13, 15, 26, 6), (17, 29, 16, 24))
    for i in range(5):
        for r in rots[i % 2]:
            x0 = x0 + x1
            x1 = _rotl(x1, r)
            x1 = x1 ^ x0
        x0 = x0 + ks[(i + 1) % 3]
        x1 = x1 + ks[(i + 2) % 3] + jnp.uint32(i + 1)
    return x0 ^ x1


def _uniform(bits):
    f = jax.lax.bitcast_convert_type((bits >> jnp.uint32(9)) | jnp.uint32(0x3F800000),
                                     jnp.float32)
    return jnp.maximum(jnp.float32(0.0), f - jnp.float32(1.0))


def _body(key_ref, x_ref, pe_ref, po_ref, pat_ref, sparse_ref, pooled_ref,
          winner_ref):
    k0 = key_ref[0].astype(jnp.uint32)
    k1 = key_ref[1].astype(jnp.uint32)

    b = pl.program_id(0)
    c = pl.program_id(1)
    g = pl.program_id(2)
    row = jax.lax.broadcasted_iota(jnp.int32, (_PB, _LW), 0)
    base = ((b * _C + c) * _NREG + g * _PB * _PW) * 5
    ctr = base + row * (5 * _PW) + pat_ref[...]
    u = _uniform(_threefry_bits(k0, k1, ctr.astype(jnp.uint32)))
    eps = jnp.float32(1e-8)
    gum = -jnp.log(-jnp.log(u + eps) + eps)

    pe = pe_ref[...]
    po = po_ref[...]

    def dot(a, b_):
        return jax.lax.dot_general(a, b_, (((1,), (0,)), ((), ())),
                                   preferred_element_type=jnp.float32)

    def dot_t(a, b_):
        return jax.lax.dot_general(a, b_, (((1,), (1,)), ((), ())),
                                   preferred_element_type=jnp.float32)

    def deinterleave(r, sel):
        hi = r.astype(jnp.bfloat16)
        rem = r - hi.astype(jnp.float32)
        lo = rem.astype(jnp.bfloat16)
        lo2 = (rem - lo.astype(jnp.float32)).astype(jnp.bfloat16)
        return dot(hi, sel) + dot(lo, sel)

    xb = x_ref[0, 0]
    r0 = xb[:, :_W]
    r1 = xb[:, _W:]
    v0 = deinterleave(r0, pe)
    v1 = deinterleave(r0, po)
    v2 = deinterleave(r1, pe)
    v3 = deinterleave(r1, po)

    m = jnp.maximum(jnp.maximum(jnp.maximum(v0, v1), jnp.maximum(v2, v3)),
                    jnp.float32(0.0))
    e0 = jnp.exp(v0 - m)
    e1 = jnp.exp(v1 - m)
    e2 = jnp.exp(v2 - m)
    e3 = jnp.exp(v3 - m)
    e4 = jnp.exp(-m)
    inv = jnp.float32(1.0) / (e0 + e1 + e2 + e3 + e4 + eps)
    probs = (e0 * inv, e1 * inv, e2 * inv, e3 * inv, e4 * inv)

    best = None
    w = None
    for k in range(5):
        s = jnp.log(probs[k] + eps) + gum[:, k * _PW:(k + 1) * _PW]
        if k == 0:
            best = s
            w = jnp.zeros((_PB, _PW), jnp.int32)
        else:
            take = s > best
            w = jnp.where(take, jnp.int32(k), w)
            best = jnp.maximum(best, s)

    sv = [jnp.where(w == k, probs[k], jnp.float32(0.0)).astype(jnp.bfloat16)
          for k in range(4)]
    sparse_ref[0, 0, :, :_W] = dot_t(sv[0], pe) + dot_t(sv[1], po)
    sparse_ref[0, 0, :, _W:] = dot_t(sv[2], pe) + dot_t(sv[3], po)
    pooled_ref[0, 0] = (e0 + e1 + e2 + e3) * inv
    winner_ref[0, 0] = w


@jax.jit
def kernel(hidden_activations):
    x = hidden_activations.reshape(_B, _C, _PH, 2 * _W)
    key_data = jax.random.key_data(
        jax.random.fold_in(jax.random.key(0), 7)).astype(jnp.int32)

    grid_spec = pltpu.PrefetchScalarGridSpec(
        num_scalar_prefetch=1,
        grid=(_B, _C, _GROWS),
        in_specs=[
            pl.BlockSpec((1, 1, _PB, 2 * _W), lambda b, c, g, k: (b, c, g, 0)),
            pl.BlockSpec((_W, _PW), lambda b, c, g, k: (0, 0)),
            pl.BlockSpec((_W, _PW), lambda b, c, g, k: (0, 0)),
            pl.BlockSpec((1, _LW), lambda b, c, g, k: (0, 0)),
        ],
        out_specs=[
            pl.BlockSpec((1, 1, _PB, 2 * _W), lambda b, c, g, k: (b, c, g, 0)),
            pl.BlockSpec((1, 1, _PB, _PW), lambda b, c, g, k: (b, c, g, 0)),
            pl.BlockSpec((1, 1, _PB, _PW), lambda b, c, g, k: (b, c, g, 0)),
        ],
    )
    sparse768, pooled, winner = pl.pallas_call(
        _body,
        grid_spec=grid_spec,
        out_shape=[
            jax.ShapeDtypeStruct((_B, _C, _PH, 2 * _W), jnp.float32),
            jax.ShapeDtypeStruct((_B, _C, _PH, _PW), jnp.float32),
            jax.ShapeDtypeStruct((_B, _C, _PH, _PW), jnp.int32),
        ],
    )(key_data, x, jnp.asarray(_PE_NP).astype(jnp.bfloat16),
      jnp.asarray(_PO_NP).astype(jnp.bfloat16), jnp.asarray(_PAT_NP))

    sparse = sparse768.reshape(_B, _C, _H, _W)
    return (sparse, pooled, winner)
